# baseline (speedup 1.0000x reference)
import functools

import jax
import jax.numpy as jnp
from jax import lax
from jax.experimental import pallas as pl
from jax.experimental.pallas import tpu as pltpu
from jax.experimental.pallas import tpu_sc as plsc

N_NODES = 10000
N_EDGES = 320000
D = 128

NC, NS = 2, 16
NW = NC * NS
E_PER_W = N_EDGES // NW
CHUNK = 125
N_CHUNKS = E_PER_W // CHUNK
N_PAD = 10240
ROWS_PER_SUB = N_PAD // NS


def _make_sc_agg(with_deg):
  mesh = plsc.VectorSubcoreMesh(core_axis_name="c", subcore_axis_name="s")

  out_type = [jax.ShapeDtypeStruct((NC, N_PAD, D), jnp.bfloat16)]
  scratch = [
      pltpu.VMEM((N_CHUNKS, CHUNK), jnp.int32),
      pltpu.VMEM((N_CHUNKS, CHUNK), jnp.int32),
      pltpu.VMEM((CHUNK, D), jnp.bfloat16),
      pltpu.VMEM((CHUNK, D), jnp.bfloat16),
      pltpu.VMEM_SHARED((N_PAD, D), jnp.bfloat16),
      pltpu.SemaphoreType.DMA,
      pltpu.SemaphoreType.DMA,
      pltpu.SemaphoreType.DMA,
      pltpu.SemaphoreType.DMA,
  ]
  if with_deg:
    out_type.append(jax.ShapeDtypeStruct((NC, N_PAD, 8), jnp.float32))
    scratch += [
        pltpu.VMEM((CHUNK, 8), jnp.float32),
        pltpu.VMEM_SHARED((N_PAD, 8), jnp.float32),
    ]

  @functools.partial(
      pl.kernel,
      out_type=out_type,
      mesh=mesh,
      scratch_types=scratch,
      compiler_params=pltpu.CompilerParams(use_tc_tiling_on_sc=False),
  )
  def agg(z_hbm, edge_hbm, zeros_hbm, *rest):
    if with_deg:
      ones_hbm, zeros8_hbm, out_hbm, deg_hbm, src_v, dst_v, rows_a, rows_b, acc, \
          gsem_a, gsem_b, ssem_a, ssem_b, ones_v, dacc = rest
    else:
      out_hbm, src_v, dst_v, rows_a, rows_b, acc, \
          gsem_a, gsem_b, ssem_a, ssem_b = rest
    cid = lax.axis_index("c")
    sid = lax.axis_index("s")
    wid = sid * NC + cid
    r0 = sid * ROWS_PER_SUB
    pltpu.sync_copy(edge_hbm.at[0, wid], src_v)
    pltpu.sync_copy(edge_hbm.at[1, wid], dst_v)
    pltpu.sync_copy(zeros_hbm.at[pl.ds(0, ROWS_PER_SUB)],
                    acc.at[pl.ds(r0, ROWS_PER_SUB)])
    if with_deg:
      pltpu.sync_copy(ones_hbm, ones_v)
      pltpu.sync_copy(zeros8_hbm, dacc.at[pl.ds(r0, ROWS_PER_SUB)])
    plsc.subcore_barrier()

    def wait_gather(rows, gsem):
      pltpu.make_async_copy(z_hbm.at[pl.ds(0, CHUNK)], rows, gsem).wait()

    def issue_scatter(rows, g, ssem):
      pltpu.async_copy(rows, acc.at[dst_v.at[g]], ssem, add=True)
      if with_deg:
        pltpu.async_copy(ones_v, dacc.at[dst_v.at[g]], ssem, add=True)

    def wait_scatter(rows, ssem):
      pltpu.make_async_copy(z_hbm.at[pl.ds(0, CHUNK)], rows, ssem).wait()
      if with_deg:
        pltpu.make_async_copy(ones_hbm, ones_v, ssem).wait()

    def gather(g, rows, gsem):
      pltpu.async_copy(z_hbm.at[src_v.at[g]], rows, gsem)

    gather(0, rows_a, gsem_a)
    wait_gather(rows_a, gsem_a)
    issue_scatter(rows_a, 0, ssem_a)
    gather(1, rows_b, gsem_b)

    def phase(g, rows_x, gsem_x, ssem_x, rows_y, ssem_y):
      wait_gather(rows_x, gsem_x)
      issue_scatter(rows_x, g, ssem_x)
      wait_scatter(rows_y, ssem_y)
      gather(g + 1, rows_y, gsem_a if rows_y is rows_a else gsem_b)

    def body(i, carry):
      g = 2 * i + 1
      phase(g, rows_b, gsem_b, ssem_b, rows_a, ssem_a)
      phase(g + 1, rows_a, gsem_a, ssem_a, rows_b, ssem_b)
      return carry

    lax.fori_loop(0, (N_CHUNKS - 2) // 2, body, 0)
    wait_gather(rows_b, gsem_b)
    issue_scatter(rows_b, N_CHUNKS - 1, ssem_b)
    wait_scatter(rows_a, ssem_a)
    wait_scatter(rows_b, ssem_b)
    plsc.subcore_barrier()
    pltpu.sync_copy(acc.at[pl.ds(r0, ROWS_PER_SUB)],
                    out_hbm.at[cid, pl.ds(r0, ROWS_PER_SUB)])
    if with_deg:
      pltpu.sync_copy(dacc.at[pl.ds(r0, ROWS_PER_SUB)],
                      deg_hbm.at[cid, pl.ds(r0, ROWS_PER_SUB)])

  return agg


_sc_agg_deg = _make_sc_agg(True)
_sc_agg_plain = _make_sc_agg(False)


BR = 2000


def _tc_self_body(z_ref, w_ref, b_ref, s_ref):
  s_ref[...] = (jnp.dot(z_ref[...], w_ref[...],
                        preferred_element_type=jnp.float32) + b_ref[...])


def _tc_layer1_body(s_ref, a_ref, d_ref, wn_ref, h_ref, hb_ref, inv_ref):
  deg = d_ref[0, :, 0] + d_ref[1, :, 0]
  inv = 1.0 / jnp.maximum(deg, 1.0)
  agg = a_ref[0].astype(jnp.float32) + a_ref[1].astype(jnp.float32)
  mean = agg * inv[:, None]
  h = s_ref[...] + jnp.dot(mean, wn_ref[...],
                           preferred_element_type=jnp.float32)
  h = jnp.maximum(h, 0.0)
  h_ref[...] = h
  hb_ref[...] = h.astype(jnp.bfloat16)
  inv_ref[...] = jnp.broadcast_to(inv[:, None], (BR, 8))


def _tc_layer2_body(s_ref, a_ref, inv_ref, wn_ref, out_ref):
  inv = inv_ref[:, 0]
  agg = a_ref[0].astype(jnp.float32) + a_ref[1].astype(jnp.float32)
  mean = agg * inv[:, None]
  out_ref[...] = s_ref[...] + jnp.dot(
      mean, wn_ref[...], preferred_element_type=jnp.float32)


def _row_spec(w):
  return pl.BlockSpec((BR, w), lambda i: (i, 0))


def _pad_spec(w):
  return pl.BlockSpec((NC, BR, w), lambda i: (0, i, 0))


def _full_spec(shape):
  return pl.BlockSpec(shape, lambda i: tuple(0 for _ in shape))


_tc_self = pl.pallas_call(
    _tc_self_body,
    grid=(N_NODES // BR,),
    in_specs=[_row_spec(D), _full_spec((D, D)), _full_spec((1, D))],
    out_specs=_row_spec(D),
    out_shape=jax.ShapeDtypeStruct((N_NODES, D), jnp.float32),
)

_tc_layer1 = pl.pallas_call(
    _tc_layer1_body,
    grid=(N_NODES // BR,),
    in_specs=[
        _row_spec(D), _pad_spec(D), _pad_spec(8),
        _full_spec((D, D)),
    ],
    out_specs=[_row_spec(D), _row_spec(D), _row_spec(8)],
    out_shape=[
        jax.ShapeDtypeStruct((N_NODES, D), jnp.float32),
        jax.ShapeDtypeStruct((N_NODES, D), jnp.bfloat16),
        jax.ShapeDtypeStruct((N_NODES, 8), jnp.float32),
    ],
)

_tc_layer2 = pl.pallas_call(
    _tc_layer2_body,
    grid=(N_NODES // BR,),
    in_specs=[
        _row_spec(D), _pad_spec(D), _row_spec(8),
        _full_spec((D, D)),
    ],
    out_specs=_row_spec(D),
    out_shape=jax.ShapeDtypeStruct((N_NODES, D), jnp.float32),
)


@jax.jit
def kernel(x, edge_index, Ws1, Wn1, b1, Ws2, Wn2, b2):
  edges = edge_index.astype(jnp.int32).reshape(2, NW, N_CHUNKS, CHUNK)
  x = x.astype(jnp.float32)
  xb = x.astype(jnp.bfloat16)
  zeros_d = jnp.zeros((ROWS_PER_SUB, D), jnp.bfloat16)
  ones_8 = jnp.ones((CHUNK, 8), jnp.float32)

  zeros_8 = jnp.zeros((ROWS_PER_SUB, 8), jnp.float32)
  s1 = _tc_self(x, Ws1, b1.reshape(1, D))
  agg1, deg = _sc_agg_deg(xb, edges, zeros_d, ones_8, zeros_8)
  h, hb, inv = _tc_layer1(s1, agg1, deg, Wn1)

  s2 = _tc_self(h, Ws2, b2.reshape(1, D))
  agg2 = _sc_agg_plain(hb, edges, zeros_d)[0]
  out = _tc_layer2(s2, agg2, inv, Wn2)
  return out


# speedup vs baseline: 1.0129x; 1.0129x over previous
import functools

import jax
import jax.numpy as jnp
from jax import lax
from jax.experimental import pallas as pl
from jax.experimental.pallas import tpu as pltpu
from jax.experimental.pallas import tpu_sc as plsc

N_NODES = 10000
N_EDGES = 320000
D = 128

NC, NS = 2, 16
NW = NC * NS
E_PER_W = N_EDGES // NW
CHUNK = 125
N_CHUNKS = E_PER_W // CHUNK
N_PAD = 10240
ROWS_PER_SUB = N_PAD // NS


def _make_sc_agg(with_deg):
  mesh = plsc.VectorSubcoreMesh(core_axis_name="c", subcore_axis_name="s")

  out_type = [jax.ShapeDtypeStruct((NC, N_PAD, D), jnp.bfloat16)]
  scratch = [
      pltpu.VMEM((N_CHUNKS, CHUNK), jnp.int32),
      pltpu.VMEM((N_CHUNKS, CHUNK), jnp.int32),
      pltpu.VMEM((CHUNK, D), jnp.bfloat16),
      pltpu.VMEM((CHUNK, D), jnp.bfloat16),
      pltpu.VMEM_SHARED((N_PAD, D), jnp.bfloat16),
      pltpu.SemaphoreType.DMA,
      pltpu.SemaphoreType.DMA,
      pltpu.SemaphoreType.DMA,
      pltpu.SemaphoreType.DMA,
  ]
  if with_deg:
    out_type.append(jax.ShapeDtypeStruct((NC, N_PAD, 8), jnp.float32))
    scratch += [
        pltpu.VMEM((CHUNK, 8), jnp.float32),
        pltpu.VMEM_SHARED((N_PAD, 8), jnp.float32),
    ]

  @functools.partial(
      pl.kernel,
      out_type=out_type,
      mesh=mesh,
      scratch_types=scratch,
      compiler_params=pltpu.CompilerParams(use_tc_tiling_on_sc=False),
  )
  def agg(z_hbm, edge_hbm, zeros_hbm, *rest):
    if with_deg:
      ones_hbm, zeros8_hbm, out_hbm, deg_hbm, src_v, dst_v, rows_a, rows_b, acc, \
          gsem_a, gsem_b, ssem_a, ssem_b, ones_v, dacc = rest
    else:
      out_hbm, src_v, dst_v, rows_a, rows_b, acc, \
          gsem_a, gsem_b, ssem_a, ssem_b = rest
    cid = lax.axis_index("c")
    sid = lax.axis_index("s")
    wid = sid * NC + cid
    r0 = sid * ROWS_PER_SUB
    pltpu.sync_copy(edge_hbm.at[0, wid], src_v)
    pltpu.sync_copy(edge_hbm.at[1, wid], dst_v)
    pltpu.sync_copy(zeros_hbm.at[pl.ds(0, ROWS_PER_SUB)],
                    acc.at[pl.ds(r0, ROWS_PER_SUB)])
    if with_deg:
      pltpu.sync_copy(ones_hbm, ones_v)
      pltpu.sync_copy(zeros8_hbm, dacc.at[pl.ds(r0, ROWS_PER_SUB)])
    plsc.subcore_barrier()

    def wait_gather(rows, gsem):
      pltpu.make_async_copy(z_hbm.at[pl.ds(0, CHUNK)], rows, gsem).wait()

    def issue_scatter(rows, g, ssem):
      pltpu.async_copy(rows, acc.at[dst_v.at[g]], ssem, add=True)
      if with_deg:
        pltpu.async_copy(ones_v, dacc.at[dst_v.at[g]], ssem, add=True)

    def wait_scatter(rows, ssem):
      pltpu.make_async_copy(z_hbm.at[pl.ds(0, CHUNK)], rows, ssem).wait()
      if with_deg:
        pltpu.make_async_copy(ones_hbm, ones_v, ssem).wait()

    def gather(g, rows, gsem):
      pltpu.async_copy(z_hbm.at[src_v.at[g]], rows, gsem)

    gather(0, rows_a, gsem_a)
    wait_gather(rows_a, gsem_a)
    issue_scatter(rows_a, 0, ssem_a)
    gather(1, rows_b, gsem_b)

    def phase(g, rows_x, gsem_x, ssem_x, rows_y, ssem_y):
      wait_gather(rows_x, gsem_x)
      issue_scatter(rows_x, g, ssem_x)
      wait_scatter(rows_y, ssem_y)
      gather(g + 1, rows_y, gsem_a if rows_y is rows_a else gsem_b)

    def body(i, carry):
      g = 2 * i + 1
      phase(g, rows_b, gsem_b, ssem_b, rows_a, ssem_a)
      phase(g + 1, rows_a, gsem_a, ssem_a, rows_b, ssem_b)
      return carry

    lax.fori_loop(0, (N_CHUNKS - 2) // 2, body, 0)
    wait_gather(rows_b, gsem_b)
    issue_scatter(rows_b, N_CHUNKS - 1, ssem_b)
    wait_scatter(rows_a, ssem_a)
    wait_scatter(rows_b, ssem_b)
    plsc.subcore_barrier()
    pltpu.sync_copy(acc.at[pl.ds(r0, ROWS_PER_SUB)],
                    out_hbm.at[cid, pl.ds(r0, ROWS_PER_SUB)])
    if with_deg:
      pltpu.sync_copy(dacc.at[pl.ds(r0, ROWS_PER_SUB)],
                      deg_hbm.at[cid, pl.ds(r0, ROWS_PER_SUB)])

  return agg


_sc_agg_deg = _make_sc_agg(True)
_sc_agg_plain = _make_sc_agg(False)


BR = 2000


def _tc_layer1_body(x_ref, a_ref, d_ref,
                    ws_ref, wn_ref, b_ref, h_ref, hb_ref, inv_ref):
  deg = d_ref[0, :, 0] + d_ref[1, :, 0]
  inv = 1.0 / jnp.maximum(deg, 1.0)
  agg = a_ref[0].astype(jnp.float32) + a_ref[1].astype(jnp.float32)
  mean = agg * inv[:, None]
  h = (jnp.dot(x_ref[...], ws_ref[...], preferred_element_type=jnp.float32)
       + jnp.dot(mean, wn_ref[...], preferred_element_type=jnp.float32)
       + b_ref[...])
  h = jnp.maximum(h, 0.0)
  h_ref[...] = h
  hb_ref[...] = h.astype(jnp.bfloat16)
  inv_ref[...] = jnp.broadcast_to(inv[:, None], (BR, 8))


def _tc_layer2_body(h_ref, a_ref, inv_ref,
                    ws_ref, wn_ref, b_ref, out_ref):
  inv = inv_ref[:, 0]
  agg = a_ref[0].astype(jnp.float32) + a_ref[1].astype(jnp.float32)
  mean = agg * inv[:, None]
  out_ref[...] = (
      jnp.dot(h_ref[...], ws_ref[...], preferred_element_type=jnp.float32)
      + jnp.dot(mean, wn_ref[...], preferred_element_type=jnp.float32)
      + b_ref[...])


def _row_spec(w):
  return pl.BlockSpec((BR, w), lambda i: (i, 0))


def _pad_spec(w):
  return pl.BlockSpec((NC, BR, w), lambda i: (0, i, 0))


def _full_spec(shape):
  return pl.BlockSpec(shape, lambda i: tuple(0 for _ in shape))


_tc_layer1 = pl.pallas_call(
    _tc_layer1_body,
    grid=(N_NODES // BR,),
    in_specs=[
        _row_spec(D), _pad_spec(D), _pad_spec(8),
        _full_spec((D, D)), _full_spec((D, D)), _full_spec((1, D)),
    ],
    out_specs=[_row_spec(D), _row_spec(D), _row_spec(8)],
    out_shape=[
        jax.ShapeDtypeStruct((N_NODES, D), jnp.float32),
        jax.ShapeDtypeStruct((N_NODES, D), jnp.bfloat16),
        jax.ShapeDtypeStruct((N_NODES, 8), jnp.float32),
    ],
)

_tc_layer2 = pl.pallas_call(
    _tc_layer2_body,
    grid=(N_NODES // BR,),
    in_specs=[
        _row_spec(D), _pad_spec(D), _row_spec(8),
        _full_spec((D, D)), _full_spec((D, D)), _full_spec((1, D)),
    ],
    out_specs=_row_spec(D),
    out_shape=jax.ShapeDtypeStruct((N_NODES, D), jnp.float32),
)


@jax.jit
def kernel(x, edge_index, Ws1, Wn1, b1, Ws2, Wn2, b2):
  edges = edge_index.astype(jnp.int32).reshape(2, NW, N_CHUNKS, CHUNK)
  x = x.astype(jnp.float32)
  xb = x.astype(jnp.bfloat16)
  zeros_d = jnp.zeros((ROWS_PER_SUB, D), jnp.bfloat16)
  ones_8 = jnp.ones((CHUNK, 8), jnp.float32)

  zeros_8 = jnp.zeros((ROWS_PER_SUB, 8), jnp.float32)
  agg1, deg = _sc_agg_deg(xb, edges, zeros_d, ones_8, zeros_8)
  h, hb, inv = _tc_layer1(x, agg1, deg, Ws1, Wn1, b1.reshape(1, D))

  agg2 = _sc_agg_plain(hb, edges, zeros_d)[0]
  out = _tc_layer2(h, agg2, inv, Ws2, Wn2, b2.reshape(1, D))
  return out


# speedup vs baseline: 1.0283x; 1.0152x over previous
import functools

import jax
import jax.numpy as jnp
from jax import lax
from jax.experimental import pallas as pl
from jax.experimental.pallas import tpu as pltpu
from jax.experimental.pallas import tpu_sc as plsc

N_NODES = 10000
N_EDGES = 320000
D = 128

NC, NS = 2, 16
NW = NC * NS
E_PER_W = N_EDGES // NW
CHUNK = 125
N_CHUNKS = E_PER_W // CHUNK
N_PAD = 10240
ROWS_PER_SUB = N_PAD // NS


def _make_sc_agg(with_deg):
  mesh = plsc.VectorSubcoreMesh(core_axis_name="c", subcore_axis_name="s")

  out_type = [jax.ShapeDtypeStruct((NC, N_PAD, D), jnp.bfloat16)]
  scratch = [
      pltpu.VMEM((N_CHUNKS, CHUNK), jnp.int32),
      pltpu.VMEM((N_CHUNKS, CHUNK), jnp.int32),
      pltpu.VMEM((CHUNK, D), jnp.bfloat16),
      pltpu.VMEM((CHUNK, D), jnp.bfloat16),
      pltpu.VMEM_SHARED((N_PAD, D), jnp.bfloat16),
      pltpu.SemaphoreType.DMA,
      pltpu.SemaphoreType.DMA,
      pltpu.SemaphoreType.DMA,
      pltpu.SemaphoreType.DMA,
  ]
  if with_deg:
    out_type.append(jax.ShapeDtypeStruct((NC, N_PAD, 8), jnp.float32))
    scratch += [
        pltpu.VMEM((CHUNK, 8), jnp.float32),
        pltpu.VMEM_SHARED((N_PAD, 8), jnp.float32),
    ]

  @functools.partial(
      pl.kernel,
      out_type=out_type,
      mesh=mesh,
      scratch_types=scratch,
      compiler_params=pltpu.CompilerParams(use_tc_tiling_on_sc=False),
  )
  def agg(z_hbm, edge_hbm, zeros_hbm, *rest):
    if with_deg:
      ones_hbm, zeros8_hbm, out_hbm, deg_hbm, src_v, dst_v, rows_a, rows_b, acc, \
          gsem_a, gsem_b, ssem_a, ssem_b, ones_v, dacc = rest
    else:
      out_hbm, src_v, dst_v, rows_a, rows_b, acc, \
          gsem_a, gsem_b, ssem_a, ssem_b = rest
    cid = lax.axis_index("c")
    sid = lax.axis_index("s")
    wid = sid * NC + cid
    r0 = sid * ROWS_PER_SUB
    pre = [pltpu.async_copy(edge_hbm.at[0, wid], src_v, gsem_a),
           pltpu.async_copy(edge_hbm.at[1, wid], dst_v, gsem_b),
           pltpu.async_copy(zeros_hbm.at[pl.ds(0, ROWS_PER_SUB)],
                            acc.at[pl.ds(r0, ROWS_PER_SUB)], ssem_a)]
    if with_deg:
      pre.append(pltpu.async_copy(ones_hbm, ones_v, ssem_b))
      pre.append(pltpu.async_copy(zeros8_hbm,
                                  dacc.at[pl.ds(r0, ROWS_PER_SUB)], ssem_b))
    for d in pre:
      d.wait()
    plsc.subcore_barrier()

    def wait_gather(rows, gsem):
      pltpu.make_async_copy(z_hbm.at[pl.ds(0, CHUNK)], rows, gsem).wait()

    def issue_scatter(rows, g, ssem):
      pltpu.async_copy(rows, acc.at[dst_v.at[g]], ssem, add=True)
      if with_deg:
        pltpu.async_copy(ones_v, dacc.at[dst_v.at[g]], ssem, add=True)

    def wait_scatter(rows, ssem):
      pltpu.make_async_copy(z_hbm.at[pl.ds(0, CHUNK)], rows, ssem).wait()
      if with_deg:
        pltpu.make_async_copy(ones_hbm, ones_v, ssem).wait()

    def gather(g, rows, gsem):
      pltpu.async_copy(z_hbm.at[src_v.at[g]], rows, gsem)

    gather(0, rows_a, gsem_a)
    wait_gather(rows_a, gsem_a)
    issue_scatter(rows_a, 0, ssem_a)
    gather(1, rows_b, gsem_b)

    def phase(g, rows_x, gsem_x, ssem_x, rows_y, ssem_y):
      wait_gather(rows_x, gsem_x)
      issue_scatter(rows_x, g, ssem_x)
      wait_scatter(rows_y, ssem_y)
      gather(g + 1, rows_y, gsem_a if rows_y is rows_a else gsem_b)

    def body(i, carry):
      g = 2 * i + 1
      phase(g, rows_b, gsem_b, ssem_b, rows_a, ssem_a)
      phase(g + 1, rows_a, gsem_a, ssem_a, rows_b, ssem_b)
      return carry

    lax.fori_loop(0, (N_CHUNKS - 2) // 2, body, 0)
    wait_gather(rows_b, gsem_b)
    issue_scatter(rows_b, N_CHUNKS - 1, ssem_b)
    wait_scatter(rows_a, ssem_a)
    wait_scatter(rows_b, ssem_b)
    plsc.subcore_barrier()
    post = [pltpu.async_copy(acc.at[pl.ds(r0, ROWS_PER_SUB)],
                             out_hbm.at[cid, pl.ds(r0, ROWS_PER_SUB)],
                             gsem_a)]
    if with_deg:
      post.append(pltpu.async_copy(dacc.at[pl.ds(r0, ROWS_PER_SUB)],
                                   deg_hbm.at[cid, pl.ds(r0, ROWS_PER_SUB)],
                                   gsem_b))
    for d in post:
      d.wait()

  return agg


_sc_agg_deg = _make_sc_agg(True)
_sc_agg_plain = _make_sc_agg(False)


BR = 2000


def _tc_layer1_body(x_ref, a_ref, d_ref,
                    ws_ref, wn_ref, b_ref, h_ref, hb_ref, inv_ref):
  deg = d_ref[0, :, 0] + d_ref[1, :, 0]
  inv = 1.0 / jnp.maximum(deg, 1.0)
  agg = a_ref[0].astype(jnp.float32) + a_ref[1].astype(jnp.float32)
  mean = agg * inv[:, None]
  h = (jnp.dot(x_ref[...], ws_ref[...], preferred_element_type=jnp.float32)
       + jnp.dot(mean, wn_ref[...], preferred_element_type=jnp.float32)
       + b_ref[...])
  h = jnp.maximum(h, 0.0)
  h_ref[...] = h
  hb_ref[...] = h.astype(jnp.bfloat16)
  inv_ref[...] = jnp.broadcast_to(inv[:, None], (BR, 8))


def _tc_layer2_body(h_ref, a_ref, inv_ref,
                    ws_ref, wn_ref, b_ref, out_ref):
  inv = inv_ref[:, 0]
  agg = a_ref[0].astype(jnp.float32) + a_ref[1].astype(jnp.float32)
  mean = agg * inv[:, None]
  out_ref[...] = (
      jnp.dot(h_ref[...], ws_ref[...], preferred_element_type=jnp.float32)
      + jnp.dot(mean, wn_ref[...], preferred_element_type=jnp.float32)
      + b_ref[...])


def _row_spec(w):
  return pl.BlockSpec((BR, w), lambda i: (i, 0))


def _pad_spec(w):
  return pl.BlockSpec((NC, BR, w), lambda i: (0, i, 0))


def _full_spec(shape):
  return pl.BlockSpec(shape, lambda i: tuple(0 for _ in shape))


_tc_layer1 = pl.pallas_call(
    _tc_layer1_body,
    grid=(N_NODES // BR,),
    in_specs=[
        _row_spec(D), _pad_spec(D), _pad_spec(8),
        _full_spec((D, D)), _full_spec((D, D)), _full_spec((1, D)),
    ],
    out_specs=[_row_spec(D), _row_spec(D), _row_spec(8)],
    out_shape=[
        jax.ShapeDtypeStruct((N_NODES, D), jnp.float32),
        jax.ShapeDtypeStruct((N_NODES, D), jnp.bfloat16),
        jax.ShapeDtypeStruct((N_NODES, 8), jnp.float32),
    ],
)

_tc_layer2 = pl.pallas_call(
    _tc_layer2_body,
    grid=(N_NODES // BR,),
    in_specs=[
        _row_spec(D), _pad_spec(D), _row_spec(8),
        _full_spec((D, D)), _full_spec((D, D)), _full_spec((1, D)),
    ],
    out_specs=_row_spec(D),
    out_shape=jax.ShapeDtypeStruct((N_NODES, D), jnp.float32),
)


@jax.jit
def kernel(x, edge_index, Ws1, Wn1, b1, Ws2, Wn2, b2):
  edges = edge_index.astype(jnp.int32).reshape(2, NW, N_CHUNKS, CHUNK)
  x = x.astype(jnp.float32)
  xb = x.astype(jnp.bfloat16)
  zeros_d = jnp.zeros((ROWS_PER_SUB, D), jnp.bfloat16)
  ones_8 = jnp.ones((CHUNK, 8), jnp.float32)

  zeros_8 = jnp.zeros((ROWS_PER_SUB, 8), jnp.float32)
  agg1, deg = _sc_agg_deg(xb, edges, zeros_d, ones_8, zeros_8)
  h, hb, inv = _tc_layer1(x, agg1, deg, Ws1, Wn1, b1.reshape(1, D))

  agg2 = _sc_agg_plain(hb, edges, zeros_d)[0]
  out = _tc_layer2(h, agg2, inv, Ws2, Wn2, b2.reshape(1, D))
  return out
